# Initial kernel scaffold; baseline (speedup 1.0000x reference)
import jax
import jax.numpy as jnp
from jax.experimental import pallas as pl


def kernel(points, feat_xz, feat_xy, feat_yz):
    raise NotImplementedError("write your pallas kernel here")


# speedup vs baseline: 31.8561x; 31.8561x over previous
import functools

import jax
import jax.numpy as jnp
from jax import lax
from jax.experimental import pallas as pl
from jax.experimental.pallas import tpu as pltpu
from jax.experimental.pallas import tpu_sc as plsc

B = 4
N = 32768
C = 128
H = 128
W = 128
G = B * N
NC = 2
NS = 16
NW = NC * NS
PW = G // NW
CHUNK = 32
NCHUNK = PW // CHUNK
L = 16

_DENOM = 1.0 + 0.1 + 10e-6


def _plane_rows_weights(u, v, boff):
    un = u / _DENOM + 0.5
    vn = v / _DENOM + 0.5
    un = jnp.where(un >= 1.0, 1.0 - 10e-6, un)
    un = jnp.where(un < 0.0, 0.0, un)
    vn = jnp.where(vn >= 1.0, 1.0 - 10e-6, vn)
    vn = jnp.where(vn < 0.0, 0.0, vn)
    iu = un * float(W - 1)
    iv = vn * float(H - 1)
    iu0 = jnp.minimum(iu.astype(jnp.int32), W - 2)
    iv0 = jnp.minimum(iv.astype(jnp.int32), H - 2)
    wu = iu - iu0.astype(jnp.float32)
    wv = iv - iv0.astype(jnp.float32)
    r00 = boff + iv0 * W + iu0
    return r00, wu, wv


@functools.partial(
    pl.kernel,
    out_type=jax.ShapeDtypeStruct((G, C), jnp.float32),
    mesh=plsc.VectorSubcoreMesh(core_axis_name="c", subcore_axis_name="s"),
    compiler_params=pltpu.CompilerParams(
        needs_layout_passes=False, use_tc_tiling_on_sc=False),
    scratch_types=[
        pltpu.VMEM((PW, 3), jnp.float32),
        pltpu.VMEM((3, 4, CHUNK, C), jnp.float32),
        pltpu.VMEM((12, CHUNK), jnp.float32),
        pltpu.VMEM((CHUNK, C), jnp.float32),
        pltpu.SemaphoreType.DMA,
    ],
)
def _sampler(tab_xz, tab_xy, tab_yz, pts, out, pts_v, r_v, w_v, acc_v, sem):
    cid = lax.axis_index("c")
    sid = lax.axis_index("s")
    wid = sid * NC + cid
    wbase = wid * PW
    pltpu.sync_copy(pts.at[pl.ds(wbase, PW)], pts_v)
    tabs = (tab_xz, tab_xy, tab_yz)
    iota = lax.iota(jnp.int32, L)

    def chunk_body(ci, carry):
        co = ci * CHUNK
        handles = []
        for grp in range(CHUNK // L):
            lp = co + grp * L + iota
            zero = jnp.zeros((L,), jnp.int32)
            x = plsc.load_gather(pts_v, [lp, zero])
            y = plsc.load_gather(pts_v, [lp, zero + 1])
            z = plsc.load_gather(pts_v, [lp, zero + 2])
            b = jnp.right_shift(wbase + lp, 15)
            boff = b * (H * W)
            for p, (u, v) in enumerate(((x, z), (x, y), (y, z))):
                r00, wu, wv = _plane_rows_weights(u, v, boff)
                corner = (
                    (0, (1.0 - wv) * (1.0 - wu)),
                    (1, (1.0 - wv) * wu),
                    (W, wv * (1.0 - wu)),
                    (W + 1, wv * wu),
                )
                for k, (off, wk) in enumerate(corner):
                    handles.append(pltpu.async_copy(
                        tabs[p].at[r00 + off],
                        r_v.at[p, k, pl.ds(grp * L, L)],
                        sem,
                    ))
                    w_v[p * 4 + k, pl.ds(grp * L, L)] = wk
        for h in handles:
            h.wait()

        def pt_body(t, inner):
            accs = [jnp.zeros((L,), jnp.float32) for _ in range(C // L)]
            tvec = jnp.full((L,), t, jnp.int32)
            for p in range(3):
                for k in range(4):
                    wb = plsc.load_gather(
                        w_v, [jnp.full((L,), p * 4 + k, jnp.int32), tvec])
                    for j in range(C // L):
                        accs[j] = accs[j] + wb * r_v[p, k, t, pl.ds(j * L, L)]
            for j in range(C // L):
                acc_v[t, pl.ds(j * L, L)] = accs[j]
            return inner

        lax.fori_loop(0, CHUNK, pt_body, 0)
        pltpu.sync_copy(acc_v, out.at[pl.ds(wbase + co, CHUNK)])
        return carry

    lax.fori_loop(0, NCHUNK, chunk_body, 0)


def kernel(points, feat_xz, feat_xy, feat_yz):
    tabs = [jnp.transpose(f, (0, 2, 3, 1)).reshape(B * H * W, C)
            for f in (feat_xz, feat_xy, feat_yz)]
    pts = points.reshape(G, 3)
    out = _sampler(tabs[0], tabs[1], tabs[2], pts)
    return out.reshape(B, N, C)


# speedup vs baseline: 32.3285x; 1.0148x over previous
import functools

import jax
import jax.numpy as jnp
from jax import lax
from jax.experimental import pallas as pl
from jax.experimental.pallas import tpu as pltpu
from jax.experimental.pallas import tpu_sc as plsc

B = 4
N = 32768
C = 128
H = 128
W = 128
G = B * N
NC = 2
NS = 16
NW = NC * NS
PW = G // NW
CHUNK = 16
NCHUNK = PW // CHUNK
L = 16

_DENOM = 1.0 + 0.1 + 10e-6


def _plane_rows_weights(u, v, boff):
    un = u / _DENOM + 0.5
    vn = v / _DENOM + 0.5
    un = jnp.where(un >= 1.0, 1.0 - 10e-6, un)
    un = jnp.where(un < 0.0, 0.0, un)
    vn = jnp.where(vn >= 1.0, 1.0 - 10e-6, vn)
    vn = jnp.where(vn < 0.0, 0.0, vn)
    iu = un * float(W - 1)
    iv = vn * float(H - 1)
    iu0 = jnp.minimum(iu.astype(jnp.int32), W - 2)
    iv0 = jnp.minimum(iv.astype(jnp.int32), H - 2)
    wu = iu - iu0.astype(jnp.float32)
    wv = iv - iv0.astype(jnp.float32)
    r00 = boff + iv0 * W + iu0
    return r00, wu, wv


@functools.partial(
    pl.kernel,
    out_type=jax.ShapeDtypeStruct((G, C), jnp.float32),
    mesh=plsc.VectorSubcoreMesh(core_axis_name="c", subcore_axis_name="s"),
    compiler_params=pltpu.CompilerParams(
        needs_layout_passes=False, use_tc_tiling_on_sc=False),
    scratch_types=[
        pltpu.VMEM((PW, 3), jnp.float32),
        pltpu.VMEM((2, 3, 4, CHUNK, C), jnp.float32),
        pltpu.VMEM((2, 12, CHUNK), jnp.float32),
        pltpu.VMEM((2, CHUNK, C), jnp.float32),
        pltpu.SemaphoreType.DMA,
        pltpu.SemaphoreType.DMA,
        pltpu.SemaphoreType.DMA,
    ],
)
def _sampler(tab_xz, tab_xy, tab_yz, pts, out,
             pts_v, r_v, w_v, acc_v, sem0, sem1, out_sem):
    cid = lax.axis_index("c")
    sid = lax.axis_index("s")
    wid = sid * NC + cid
    wbase = wid * PW
    pltpu.sync_copy(pts.at[pl.ds(wbase, PW)], pts_v)
    tabs = (tab_xz, tab_xy, tab_yz)
    iota = lax.iota(jnp.int32, L)
    sems = (sem0, sem1)

    def issue(ci, buf):
        co = ci * CHUNK
        sem = sems[buf]
        for grp in range(CHUNK // L):
            lp = co + grp * L + iota
            x = plsc.load_gather(pts_v, [lp, jnp.zeros((L,), jnp.int32)])
            y = plsc.load_gather(pts_v, [lp, jnp.full((L,), 1, jnp.int32)])
            z = plsc.load_gather(pts_v, [lp, jnp.full((L,), 2, jnp.int32)])
            b = jnp.right_shift(wbase + lp, 15)
            boff = b * (H * W)
            for p, (u, v) in enumerate(((x, z), (x, y), (y, z))):
                r00, wu, wv = _plane_rows_weights(u, v, boff)
                corner = (
                    (0, (1.0 - wv) * (1.0 - wu)),
                    (1, (1.0 - wv) * wu),
                    (W, wv * (1.0 - wu)),
                    (W + 1, wv * wu),
                )
                for k, (off, wk) in enumerate(corner):
                    pltpu.async_copy(
                        tabs[p].at[r00 + off],
                        r_v.at[buf, p, k, pl.ds(grp * L, L)],
                        sem,
                    )
                    w_v[buf, p * 4 + k, pl.ds(grp * L, L)] = wk

    def wait_rows(buf):
        sem = sems[buf]
        for p in range(3):
            for k in range(4):
                for grp in range(CHUNK // L):
                    pltpu.make_async_copy(
                        tab_xz.at[pl.ds(0, L)],
                        r_v.at[buf, p, k, pl.ds(grp * L, L)],
                        sem,
                    ).wait()

    def wait_out(buf):
        pltpu.make_async_copy(
            acc_v.at[buf], out.at[pl.ds(0, CHUNK)], out_sem).wait()

    def accumulate(ci, buf):
        def pt_body(t, inner):
            accs = [jnp.zeros((L,), jnp.float32) for _ in range(C // L)]
            tvec = jnp.full((L,), t, jnp.int32)
            for p in range(3):
                for k in range(4):
                    wb = plsc.load_gather(
                        w_v, [jnp.full((L,), buf, jnp.int32),
                              jnp.full((L,), p * 4 + k, jnp.int32), tvec])
                    for j in range(C // L):
                        accs[j] = accs[j] + wb * r_v[buf, p, k, t,
                                                     pl.ds(j * L, L)]
            for j in range(C // L):
                acc_v[buf, t, pl.ds(j * L, L)] = accs[j]
            return inner

        lax.fori_loop(0, CHUNK, pt_body, 0)
        pltpu.async_copy(acc_v.at[buf],
                         out.at[pl.ds(wbase + ci * CHUNK, CHUNK)], out_sem)

    issue(0, 0)

    def pair_body(i, carry):
        c0 = i * 2
        issue(c0 + 1, 1)
        wait_rows(0)

        @pl.when(i > 0)
        def _():
            wait_out(0)

        accumulate(c0, 0)

        @pl.when(i < NCHUNK // 2 - 1)
        def _():
            issue(c0 + 2, 0)

        wait_rows(1)

        @pl.when(i > 0)
        def _():
            wait_out(1)

        accumulate(c0 + 1, 1)
        return carry

    lax.fori_loop(0, NCHUNK // 2, pair_body, 0)
    wait_out(0)
    wait_out(1)


def kernel(points, feat_xz, feat_xy, feat_yz):
    tabs = [jnp.transpose(f, (0, 2, 3, 1)).reshape(B * H * W, C)
            for f in (feat_xz, feat_xy, feat_yz)]
    pts = points.reshape(G, 3)
    out = _sampler(tabs[0], tabs[1], tabs[2], pts)
    return out.reshape(B, N, C)
